# Initial kernel scaffold; baseline (speedup 1.0000x reference)
import jax
import jax.numpy as jnp
from jax.experimental import pallas as pl


def kernel(x, segment_ids, weight, bias):
    raise NotImplementedError("write your pallas kernel here")


# speedup vs baseline: 6.2093x; 6.2093x over previous
import jax
import jax.numpy as jnp
from jax.experimental import pallas as pl

N = 320000
C = 128
NUM_SEGMENTS = 16
EPS = 1e-6
BR = 4000


def _stats_kernel(x_ref, seg_ref, sums_ref, counts_ref):
    @pl.when(pl.program_id(0) == 0)
    def _():
        sums_ref[...] = jnp.zeros_like(sums_ref)
        counts_ref[...] = jnp.zeros_like(counts_ref)

    xb = x_ref[...]
    seg = seg_ref[...]
    iota = jax.lax.broadcasted_iota(jnp.int32, (BR, NUM_SEGMENTS), 1)
    onehot = (seg == iota).astype(jnp.float32)
    xcat = jnp.concatenate([xb, xb * xb], axis=1)
    part = jax.lax.dot_general(
        onehot, xcat, (((0,), (0,)), ((), ())),
        preferred_element_type=jnp.float32,
        precision=jax.lax.Precision.HIGHEST)
    sums_ref[...] += part
    counts_ref[...] += jnp.sum(onehot, axis=0)[:, None]


def _norm_kernel(x_ref, seg_ref, stats_ref, counts_ref, w_ref, b_ref, out_ref):
    cnt = jnp.maximum(counts_ref[...], 1.0)
    inv = 1.0 / cnt
    mean = stats_ref[:, :C] * inv
    msq = stats_ref[:, C:] * inv
    var = msq - mean * mean
    instd = jax.lax.rsqrt(var + EPS)
    scale = instd * w_ref[...]
    shift = b_ref[...] - mean * scale

    seg = seg_ref[...]
    iota = jax.lax.broadcasted_iota(jnp.int32, (BR, NUM_SEGMENTS), 1)
    onehot = (seg == iota).astype(jnp.float32)
    S = jax.lax.dot_general(
        onehot, scale, (((1,), (0,)), ((), ())),
        preferred_element_type=jnp.float32,
        precision=jax.lax.Precision.HIGHEST)
    T = jax.lax.dot_general(
        onehot, shift, (((1,), (0,)), ((), ())),
        preferred_element_type=jnp.float32,
        precision=jax.lax.Precision.HIGHEST)
    out_ref[...] = x_ref[...] * S + T


def kernel(x, segment_ids, weight, bias):
    seg2d = segment_ids.astype(jnp.int32).reshape(N, 1)
    grid = (N // BR,)

    stats, counts = pl.pallas_call(
        _stats_kernel,
        grid=grid,
        in_specs=[
            pl.BlockSpec((BR, C), lambda i: (i, 0)),
            pl.BlockSpec((BR, 1), lambda i: (i, 0)),
        ],
        out_specs=[
            pl.BlockSpec((NUM_SEGMENTS, 2 * C), lambda i: (0, 0)),
            pl.BlockSpec((NUM_SEGMENTS, 1), lambda i: (0, 0)),
        ],
        out_shape=[
            jax.ShapeDtypeStruct((NUM_SEGMENTS, 2 * C), jnp.float32),
            jax.ShapeDtypeStruct((NUM_SEGMENTS, 1), jnp.float32),
        ],
    )(x, seg2d)

    out = pl.pallas_call(
        _norm_kernel,
        grid=grid,
        in_specs=[
            pl.BlockSpec((BR, C), lambda i: (i, 0)),
            pl.BlockSpec((BR, 1), lambda i: (i, 0)),
            pl.BlockSpec((NUM_SEGMENTS, 2 * C), lambda i: (0, 0)),
            pl.BlockSpec((NUM_SEGMENTS, 1), lambda i: (0, 0)),
            pl.BlockSpec((1, C), lambda i: (0, 0)),
            pl.BlockSpec((1, C), lambda i: (0, 0)),
        ],
        out_specs=pl.BlockSpec((BR, C), lambda i: (i, 0)),
        out_shape=jax.ShapeDtypeStruct((N, C), jnp.float32),
    )(x, seg2d, stats, counts, weight, bias)
    return out


# speedup vs baseline: 8.5111x; 1.3707x over previous
import jax
import jax.numpy as jnp
from jax.experimental import pallas as pl

N = 320000
C = 128
NUM_SEGMENTS = 16
EPS = 1e-6
BR = 4000


def _stats_kernel(x_ref, seg_ref, sums_ref, counts_ref):
    @pl.when(pl.program_id(0) == 0)
    def _():
        sums_ref[...] = jnp.zeros_like(sums_ref)
        counts_ref[...] = jnp.zeros_like(counts_ref)

    xb = x_ref[...]
    seg = seg_ref[...]
    iota = jax.lax.broadcasted_iota(jnp.int32, (BR, NUM_SEGMENTS), 1)
    onehot = (seg == iota).astype(jnp.float32)
    ps = jax.lax.dot_general(
        onehot, xb, (((0,), (0,)), ((), ())),
        preferred_element_type=jnp.float32)
    pq = jax.lax.dot_general(
        onehot, xb * xb, (((0,), (0,)), ((), ())),
        preferred_element_type=jnp.float32)
    sums_ref[:, :C] += ps
    sums_ref[:, C:] += pq
    counts_ref[...] += jnp.sum(onehot, axis=0)[:, None]


def _norm_kernel(x_ref, seg_ref, stats_ref, counts_ref, w_ref, b_ref, out_ref):
    cnt = jnp.maximum(counts_ref[...], 1.0)
    inv = 1.0 / cnt
    mean = stats_ref[:, :C] * inv
    msq = stats_ref[:, C:] * inv
    var = msq - mean * mean
    instd = jax.lax.rsqrt(var + EPS)
    scale = instd * w_ref[...]
    shift = b_ref[...] - mean * scale

    seg = seg_ref[...]
    iota = jax.lax.broadcasted_iota(jnp.int32, (BR, NUM_SEGMENTS), 1)
    onehot = (seg == iota).astype(jnp.float32)
    st = jnp.concatenate([scale, shift], axis=1)
    ST = jax.lax.dot_general(
        onehot, st, (((1,), (0,)), ((), ())),
        preferred_element_type=jnp.float32)
    out_ref[...] = x_ref[...] * ST[:, :C] + ST[:, C:]


def kernel(x, segment_ids, weight, bias):
    seg2d = segment_ids.astype(jnp.int32).reshape(N, 1)
    grid = (N // BR,)

    stats, counts = pl.pallas_call(
        _stats_kernel,
        grid=grid,
        in_specs=[
            pl.BlockSpec((BR, C), lambda i: (i, 0)),
            pl.BlockSpec((BR, 1), lambda i: (i, 0)),
        ],
        out_specs=[
            pl.BlockSpec((NUM_SEGMENTS, 2 * C), lambda i: (0, 0)),
            pl.BlockSpec((NUM_SEGMENTS, 1), lambda i: (0, 0)),
        ],
        out_shape=[
            jax.ShapeDtypeStruct((NUM_SEGMENTS, 2 * C), jnp.float32),
            jax.ShapeDtypeStruct((NUM_SEGMENTS, 1), jnp.float32),
        ],
    )(x, seg2d)

    out = pl.pallas_call(
        _norm_kernel,
        grid=grid,
        in_specs=[
            pl.BlockSpec((BR, C), lambda i: (i, 0)),
            pl.BlockSpec((BR, 1), lambda i: (i, 0)),
            pl.BlockSpec((NUM_SEGMENTS, 2 * C), lambda i: (0, 0)),
            pl.BlockSpec((NUM_SEGMENTS, 1), lambda i: (0, 0)),
            pl.BlockSpec((1, C), lambda i: (0, 0)),
            pl.BlockSpec((1, C), lambda i: (0, 0)),
        ],
        out_specs=pl.BlockSpec((BR, C), lambda i: (i, 0)),
        out_shape=jax.ShapeDtypeStruct((N, C), jnp.float32),
    )(x, seg2d, stats, counts, weight, bias)
    return out


# speedup vs baseline: 15.9562x; 1.8748x over previous
import jax
import jax.numpy as jnp
from jax.experimental import pallas as pl

N = 320000
C = 128
NUM_SEGMENTS = 16
EPS = 1e-6
BR = 4000


def _onehot_t(seg_ref):
    seg = seg_ref[0, 0, :]
    segb = jnp.broadcast_to(seg[None, :], (NUM_SEGMENTS, BR))
    tid = jax.lax.broadcasted_iota(jnp.int32, (NUM_SEGMENTS, BR), 0)
    return (segb == tid).astype(jnp.float32)


def _stats_kernel(x_ref, seg_ref, sums_ref, counts_ref):
    @pl.when(pl.program_id(0) == 0)
    def _():
        sums_ref[...] = jnp.zeros_like(sums_ref)
        counts_ref[...] = jnp.zeros_like(counts_ref)

    xb = x_ref[...]
    oh = _onehot_t(seg_ref)
    ps = jax.lax.dot_general(
        oh, xb, (((1,), (0,)), ((), ())),
        preferred_element_type=jnp.float32)
    pq = jax.lax.dot_general(
        oh, xb * xb, (((1,), (0,)), ((), ())),
        preferred_element_type=jnp.float32)
    sums_ref[:, :C] += ps
    sums_ref[:, C:] += pq
    counts_ref[...] += jnp.sum(oh, axis=1)[:, None]


def _norm_kernel(x_ref, seg_ref, stats_ref, counts_ref, w_ref, b_ref, out_ref):
    cnt = jnp.maximum(counts_ref[...], 1.0)
    inv = 1.0 / cnt
    mean = stats_ref[:, :C] * inv
    msq = stats_ref[:, C:] * inv
    var = msq - mean * mean
    instd = jax.lax.rsqrt(var + EPS)
    scale = instd * w_ref[...]
    shift = b_ref[...] - mean * scale
    st = jnp.concatenate([scale, shift], axis=1)

    oh = _onehot_t(seg_ref)
    ST = jax.lax.dot_general(
        oh, st, (((0,), (0,)), ((), ())),
        preferred_element_type=jnp.float32)
    out_ref[...] = x_ref[...] * ST[:, :C] + ST[:, C:]


def kernel(x, segment_ids, weight, bias):
    seg3d = segment_ids.astype(jnp.int32).reshape(N // BR, 1, BR)
    grid = (N // BR,)

    stats, counts = pl.pallas_call(
        _stats_kernel,
        grid=grid,
        in_specs=[
            pl.BlockSpec((BR, C), lambda i: (i, 0)),
            pl.BlockSpec((1, 1, BR), lambda i: (i, 0, 0)),
        ],
        out_specs=[
            pl.BlockSpec((NUM_SEGMENTS, 2 * C), lambda i: (0, 0)),
            pl.BlockSpec((NUM_SEGMENTS, 1), lambda i: (0, 0)),
        ],
        out_shape=[
            jax.ShapeDtypeStruct((NUM_SEGMENTS, 2 * C), jnp.float32),
            jax.ShapeDtypeStruct((NUM_SEGMENTS, 1), jnp.float32),
        ],
    )(x, seg3d)

    out = pl.pallas_call(
        _norm_kernel,
        grid=grid,
        in_specs=[
            pl.BlockSpec((BR, C), lambda i: (i, 0)),
            pl.BlockSpec((1, 1, BR), lambda i: (i, 0, 0)),
            pl.BlockSpec((NUM_SEGMENTS, 2 * C), lambda i: (0, 0)),
            pl.BlockSpec((NUM_SEGMENTS, 1), lambda i: (0, 0)),
            pl.BlockSpec((1, C), lambda i: (0, 0)),
            pl.BlockSpec((1, C), lambda i: (0, 0)),
        ],
        out_specs=pl.BlockSpec((BR, C), lambda i: (i, 0)),
        out_shape=jax.ShapeDtypeStruct((N, C), jnp.float32),
    )(x, seg3d, stats, counts, weight, bias)
    return out


# speedup vs baseline: 20.2995x; 1.2722x over previous
import jax
import jax.numpy as jnp
from jax.experimental import pallas as pl

N = 320000
C = 128
NUM_SEGMENTS = 16
EPS = 1e-6
BR = 8000


def _onehot_t(seg_ref):
    seg = seg_ref[0, 0, :]
    segb = jnp.broadcast_to(seg[None, :], (NUM_SEGMENTS, BR))
    tid = jax.lax.broadcasted_iota(jnp.int32, (NUM_SEGMENTS, BR), 0)
    return (segb == tid).astype(jnp.float32)


def _stats_kernel(x_ref, seg_ref, sums_ref, counts_ref):
    @pl.when(pl.program_id(0) == 0)
    def _():
        sums_ref[...] = jnp.zeros_like(sums_ref)
        counts_ref[...] = jnp.zeros_like(counts_ref)

    xb = x_ref[...]
    oh = _onehot_t(seg_ref)
    ps = jax.lax.dot_general(
        oh, xb, (((1,), (0,)), ((), ())),
        preferred_element_type=jnp.float32)
    pq = jax.lax.dot_general(
        oh, xb * xb, (((1,), (0,)), ((), ())),
        preferred_element_type=jnp.float32)
    sums_ref[:, :C] += ps
    sums_ref[:, C:] += pq
    counts_ref[...] += jnp.sum(oh, axis=1)[:, None]


def _norm_kernel(x_ref, seg_ref, stats_ref, counts_ref, w_ref, b_ref, out_ref):
    cnt = jnp.maximum(counts_ref[...], 1.0)
    inv = 1.0 / cnt
    mean = stats_ref[:, :C] * inv
    msq = stats_ref[:, C:] * inv
    var = msq - mean * mean
    instd = jax.lax.rsqrt(var + EPS)
    scale = instd * w_ref[...]
    shift = b_ref[...] - mean * scale
    st = jnp.concatenate([scale, shift], axis=1)

    oh = _onehot_t(seg_ref)
    ST = jax.lax.dot_general(
        oh, st, (((0,), (0,)), ((), ())),
        preferred_element_type=jnp.float32)
    out_ref[...] = x_ref[...] * ST[:, :C] + ST[:, C:]


def kernel(x, segment_ids, weight, bias):
    seg3d = segment_ids.astype(jnp.int32).reshape(N // BR, 1, BR)
    grid = (N // BR,)

    stats, counts = pl.pallas_call(
        _stats_kernel,
        grid=grid,
        in_specs=[
            pl.BlockSpec((BR, C), lambda i: (i, 0)),
            pl.BlockSpec((1, 1, BR), lambda i: (i, 0, 0)),
        ],
        out_specs=[
            pl.BlockSpec((NUM_SEGMENTS, 2 * C), lambda i: (0, 0)),
            pl.BlockSpec((NUM_SEGMENTS, 1), lambda i: (0, 0)),
        ],
        out_shape=[
            jax.ShapeDtypeStruct((NUM_SEGMENTS, 2 * C), jnp.float32),
            jax.ShapeDtypeStruct((NUM_SEGMENTS, 1), jnp.float32),
        ],
    )(x, seg3d)

    out = pl.pallas_call(
        _norm_kernel,
        grid=grid,
        in_specs=[
            pl.BlockSpec((BR, C), lambda i: (i, 0)),
            pl.BlockSpec((1, 1, BR), lambda i: (i, 0, 0)),
            pl.BlockSpec((NUM_SEGMENTS, 2 * C), lambda i: (0, 0)),
            pl.BlockSpec((NUM_SEGMENTS, 1), lambda i: (0, 0)),
            pl.BlockSpec((1, C), lambda i: (0, 0)),
            pl.BlockSpec((1, C), lambda i: (0, 0)),
        ],
        out_specs=pl.BlockSpec((BR, C), lambda i: (i, 0)),
        out_shape=jax.ShapeDtypeStruct((N, C), jnp.float32),
    )(x, seg3d, stats, counts, weight, bias)
    return out


# speedup vs baseline: 22.2723x; 1.0972x over previous
import jax
import jax.numpy as jnp
from jax.experimental import pallas as pl

N = 320000
C = 128
NUM_SEGMENTS = 16
EPS = 1e-6
BR = 16000


def _onehot_t(seg_ref):
    seg = seg_ref[0, 0, :]
    segb = jnp.broadcast_to(seg[None, :], (NUM_SEGMENTS, BR))
    tid = jax.lax.broadcasted_iota(jnp.int32, (NUM_SEGMENTS, BR), 0)
    return (segb == tid).astype(jnp.float32)


def _stats_kernel(x_ref, seg_ref, sums_ref, counts_ref):
    @pl.when(pl.program_id(0) == 0)
    def _():
        sums_ref[...] = jnp.zeros_like(sums_ref)
        counts_ref[...] = jnp.zeros_like(counts_ref)

    xb = x_ref[...]
    oh = _onehot_t(seg_ref)
    ps = jax.lax.dot_general(
        oh, xb, (((1,), (0,)), ((), ())),
        preferred_element_type=jnp.float32)
    pq = jax.lax.dot_general(
        oh, xb * xb, (((1,), (0,)), ((), ())),
        preferred_element_type=jnp.float32)
    sums_ref[:, :C] += ps
    sums_ref[:, C:] += pq
    counts_ref[...] += jnp.sum(oh, axis=1)[:, None]


def _norm_kernel(x_ref, seg_ref, stats_ref, counts_ref, w_ref, b_ref, out_ref):
    cnt = jnp.maximum(counts_ref[...], 1.0)
    inv = 1.0 / cnt
    mean = stats_ref[:, :C] * inv
    msq = stats_ref[:, C:] * inv
    var = msq - mean * mean
    instd = jax.lax.rsqrt(var + EPS)
    scale = instd * w_ref[...]
    shift = b_ref[...] - mean * scale
    st = jnp.concatenate([scale, shift], axis=1)

    oh = _onehot_t(seg_ref)
    ST = jax.lax.dot_general(
        oh, st, (((0,), (0,)), ((), ())),
        preferred_element_type=jnp.float32)
    out_ref[...] = x_ref[...] * ST[:, :C] + ST[:, C:]


def kernel(x, segment_ids, weight, bias):
    seg3d = segment_ids.astype(jnp.int32).reshape(N // BR, 1, BR)
    grid = (N // BR,)

    stats, counts = pl.pallas_call(
        _stats_kernel,
        grid=grid,
        in_specs=[
            pl.BlockSpec((BR, C), lambda i: (i, 0)),
            pl.BlockSpec((1, 1, BR), lambda i: (i, 0, 0)),
        ],
        out_specs=[
            pl.BlockSpec((NUM_SEGMENTS, 2 * C), lambda i: (0, 0)),
            pl.BlockSpec((NUM_SEGMENTS, 1), lambda i: (0, 0)),
        ],
        out_shape=[
            jax.ShapeDtypeStruct((NUM_SEGMENTS, 2 * C), jnp.float32),
            jax.ShapeDtypeStruct((NUM_SEGMENTS, 1), jnp.float32),
        ],
    )(x, seg3d)

    out = pl.pallas_call(
        _norm_kernel,
        grid=grid,
        in_specs=[
            pl.BlockSpec((BR, C), lambda i: (i, 0)),
            pl.BlockSpec((1, 1, BR), lambda i: (i, 0, 0)),
            pl.BlockSpec((NUM_SEGMENTS, 2 * C), lambda i: (0, 0)),
            pl.BlockSpec((NUM_SEGMENTS, 1), lambda i: (0, 0)),
            pl.BlockSpec((1, C), lambda i: (0, 0)),
            pl.BlockSpec((1, C), lambda i: (0, 0)),
        ],
        out_specs=pl.BlockSpec((BR, C), lambda i: (i, 0)),
        out_shape=jax.ShapeDtypeStruct((N, C), jnp.float32),
    )(x, seg3d, stats, counts, weight, bias)
    return out
